# baseline (speedup 1.0000x reference)
import functools

import jax
import jax.numpy as jnp
from jax import lax
from jax.experimental import pallas as pl
from jax.experimental.pallas import tpu as pltpu
from jax.experimental.pallas import tpu_sc as plsc

N = 10000
E = 320000
D = 128
H = 32
C = 64
EPS = 0.1
BN_EPS = 1e-5

NUM_CORES = 2
NUM_SUBCORES = 16
NUM_TILES = NUM_CORES * NUM_SUBCORES

EDGES_PER_TILE = E // NUM_TILES
CHUNK = 125
CHUNKS_PER_TILE = EDGES_PER_TILE // CHUNK
NBUF = 8
PREFETCH = 4

NP = 10016
FLAT = NP * H // 128
FREAL = N * H // 128
ROWS_Z = NP // NUM_SUBCORES


def _sc_segment_sum(y, ei):
  mesh = plsc.VectorSubcoreMesh(
      core_axis_name="c", subcore_axis_name="s",
      num_cores=NUM_CORES, num_subcores=NUM_SUBCORES)

  @functools.partial(
      pl.kernel,
      out_type=jax.ShapeDtypeStruct((NUM_CORES, NP, H), jnp.float32),
      mesh=mesh,
      scratch_types=[
          pltpu.VMEM((CHUNKS_PER_TILE, CHUNK), jnp.int32),
          pltpu.VMEM((CHUNKS_PER_TILE, CHUNK), jnp.int32),
          pltpu.VMEM((NBUF, CHUNK, H), jnp.float32),
          pltpu.VMEM((ROWS_Z, H), jnp.float32),
          pltpu.VMEM_SHARED((NP, H), jnp.float32),
          pltpu.SemaphoreType.DMA((NBUF,)),
          pltpu.SemaphoreType.DMA((NBUF,)),
          pltpu.SemaphoreType.DMA,
      ],
      compiler_params=pltpu.CompilerParams(use_tc_tiling_on_sc=False),
  )
  def seg_kernel(y_hbm, ei_hbm, out_hbm,
                 src_v, dst_v, rows, zbuf, acc, gsem, ssem, isem):
    c = lax.axis_index("c")
    s = lax.axis_index("s")
    wid = c * NUM_SUBCORES + s

    i_src = pltpu.async_copy(ei_hbm.at[0].at[wid], src_v, isem)
    i_dst = pltpu.async_copy(ei_hbm.at[1].at[wid], dst_v, isem)

    zero16 = jnp.zeros((16,), jnp.float32)

    @pl.loop(0, ROWS_Z)
    def _(r):
      zbuf[r, pl.ds(0, 16)] = zero16
      zbuf[r, pl.ds(16, 16)] = zero16

    row0 = s * ROWS_Z
    pltpu.sync_copy(zbuf, acc.at[pl.ds(row0, ROWS_Z)])

    i_src.wait()
    i_dst.wait()

    def fire_gather(i, b):
      pltpu.async_copy(y_hbm.at[src_v.at[i]], rows.at[b], gsem.at[b])

    def fire_scatter(i, b):
      pltpu.async_copy(rows.at[b], acc.at[dst_v.at[i]], ssem.at[b], add=True)

    def wait_gather(b):
      pltpu.make_async_copy(y_hbm.at[src_v.at[0]], rows.at[b],
                            gsem.at[b]).wait()

    def wait_scatter(b):
      pltpu.make_async_copy(rows.at[b], acc.at[dst_v.at[0]],
                            ssem.at[b]).wait()

    for k in range(PREFETCH):
      fire_gather(k, k)

    plsc.subcore_barrier()

    for i in range(PREFETCH):
      fire_gather(i + PREFETCH, (i + PREFETCH) % NBUF)
      wait_gather(i % NBUF)
      fire_scatter(i, i % NBUF)

    @pl.loop(PREFETCH, CHUNKS_PER_TILE - PREFETCH)
    def _(i):
      b = lax.rem(i, NBUF)
      bn = lax.rem(i + PREFETCH, NBUF)
      wait_scatter(bn)
      fire_gather(i + PREFETCH, bn)
      wait_gather(b)
      fire_scatter(i, b)

    for i in range(CHUNKS_PER_TILE - PREFETCH, CHUNKS_PER_TILE):
      b = i % NBUF
      wait_gather(b)
      fire_scatter(i, b)

    for i in range(CHUNKS_PER_TILE - NBUF, CHUNKS_PER_TILE):
      wait_scatter(i % NBUF)

    plsc.subcore_barrier()
    pltpu.sync_copy(acc.at[pl.ds(row0, ROWS_Z)],
                    out_hbm.at[c].at[pl.ds(row0, ROWS_Z)])

  return seg_kernel(y, ei)


def _tc_project(x, W0, ei):
  def body(x_ref, w_ref, ei_ref, o_ref):
    del ei_ref
    o_ref[...] = jnp.dot(x_ref[...], w_ref[...],
                         preferred_element_type=jnp.float32,
                         precision=lax.Precision.HIGHEST)
  grid = 10
  br = N // grid
  return pl.pallas_call(
      body,
      grid=(grid,),
      in_specs=[pl.BlockSpec((br, D), lambda i: (i, 0)),
                pl.BlockSpec((D, H), lambda i: (0, 0)),
                pl.BlockSpec(memory_space=pltpu.MemorySpace.HBM)],
      out_specs=pl.BlockSpec((br, H), lambda i: (i, 0)),
      out_shape=jax.ShapeDtypeStruct((NP, H), jnp.float32))(x, W0, ei)


def _row_mask(v):
  rows = lax.broadcasted_iota(jnp.int32, (FLAT, 1), 0)
  return jnp.where(rows < FREAL, v, 0.0)


def _tc_combine_project(acc, y, Wbig, bias128):
  def body(a_ref, y_ref, w_ref, b_ref, o_ref):
    pre = a_ref[0] + a_ref[1] + (1.0 + EPS) * y_ref[...]
    h = _row_mask(jnp.maximum(pre + b_ref[...], 0.0))
    o_ref[...] = jnp.dot(h, w_ref[...],
                         preferred_element_type=jnp.float32,
                         precision=lax.Precision.HIGHEST)
  return pl.pallas_call(
      body, out_shape=jax.ShapeDtypeStruct((FLAT, 128), jnp.float32))(
          acc, y, Wbig, bias128)


def _tc_finalize(acc, y, WoutBig, bias128):
  def body(a_ref, y_ref, w_ref, b_ref, o_ref):
    pre = a_ref[0] + a_ref[1] + (1.0 + EPS) * y_ref[...]
    h = _row_mask(jnp.maximum(pre + b_ref[...], 0.0))
    pooled = jnp.sum(h, axis=0, keepdims=True)
    o_ref[...] = jnp.dot(pooled, w_ref[...],
                         preferred_element_type=jnp.float32,
                         precision=lax.Precision.HIGHEST)
  return pl.pallas_call(
      body, out_shape=jax.ShapeDtypeStruct((1, C), jnp.float32))(
          acc, y, WoutBig, bias128)


def kernel(x, edge_index, W0, W1, W_out, bn_scale, bn_bias, bn_mean, bn_var):
  ei = edge_index.astype(jnp.int32).reshape(
      2, NUM_TILES, CHUNKS_PER_TILE, CHUNK)
  s = (bn_scale * lax.rsqrt(bn_var + BN_EPS)).astype(jnp.float32)
  cvec = bn_bias - bn_mean * s
  bias128 = jnp.tile(cvec.reshape(1, H), (1, 4))
  W0f = W0 * s[None, :]
  W1f = W1 * s[None, :]
  W1big = jnp.kron(jnp.eye(4, dtype=jnp.float32), W1f)
  WoutBig = jnp.tile(W_out, (4, 1))

  y0p = _tc_project(x, W0f, ei)
  acc0 = _sc_segment_sum(y0p, ei)
  y1 = _tc_combine_project(
      acc0.reshape(NUM_CORES, FLAT, 128), y0p.reshape(FLAT, 128),
      W1big, bias128)
  acc1 = _sc_segment_sum(y1.reshape(NP, H), ei)
  out = _tc_finalize(
      acc1.reshape(NUM_CORES, FLAT, 128), y1, WoutBig, bias128)
  return out.reshape(C)


# speedup vs baseline: 1.0349x; 1.0349x over previous
import functools

import jax
import jax.numpy as jnp
from jax import lax
from jax.experimental import pallas as pl
from jax.experimental.pallas import tpu as pltpu
from jax.experimental.pallas import tpu_sc as plsc

N = 10000
E = 320000
D = 128
H = 32
C = 64
EPS = 0.1
BN_EPS = 1e-5

NUM_CORES = 2
NUM_SUBCORES = 16
NUM_TILES = NUM_CORES * NUM_SUBCORES

EDGES_PER_TILE = E // NUM_TILES
CHUNK = 125
CHUNKS_PER_TILE = EDGES_PER_TILE // CHUNK
NBUF = 8
PREFETCH = 4

NP = 10016
FLAT = NP * H // 128
FREAL = N * H // 128
ROWS_Z = NP // NUM_SUBCORES


def _sc_segment_sum(y, ei):
  mesh = plsc.VectorSubcoreMesh(
      core_axis_name="c", subcore_axis_name="s",
      num_cores=NUM_CORES, num_subcores=NUM_SUBCORES)

  @functools.partial(
      pl.kernel,
      out_type=jax.ShapeDtypeStruct((NUM_CORES, NP, H), jnp.float32),
      mesh=mesh,
      scratch_types=[
          pltpu.VMEM((CHUNKS_PER_TILE, CHUNK), jnp.int32),
          pltpu.VMEM((CHUNKS_PER_TILE, CHUNK), jnp.int32),
          pltpu.VMEM((NBUF, CHUNK, H), jnp.float32),
          pltpu.VMEM((ROWS_Z, H), jnp.float32),
          pltpu.VMEM_SHARED((NP, H), jnp.float32),
          pltpu.SemaphoreType.DMA((NBUF,)),
          pltpu.SemaphoreType.DMA((NBUF,)),
          pltpu.SemaphoreType.DMA,
      ],
      compiler_params=pltpu.CompilerParams(use_tc_tiling_on_sc=False),
  )
  def seg_kernel(y_hbm, ei_hbm, out_hbm,
                 src_v, dst_v, rows, zbuf, acc, gsem, ssem, isem):
    c = lax.axis_index("c")
    s = lax.axis_index("s")
    wid = c * NUM_SUBCORES + s

    i_src = pltpu.async_copy(ei_hbm.at[0].at[wid], src_v, isem)
    i_dst = pltpu.async_copy(ei_hbm.at[1].at[wid], dst_v, isem)

    zero16 = jnp.zeros((16,), jnp.float32)

    @pl.loop(0, ROWS_Z)
    def _(r):
      zbuf[r, pl.ds(0, 16)] = zero16
      zbuf[r, pl.ds(16, 16)] = zero16

    row0 = s * ROWS_Z
    pltpu.sync_copy(zbuf, acc.at[pl.ds(row0, ROWS_Z)])

    i_src.wait()
    i_dst.wait()

    def fire_gather(i, b):
      pltpu.async_copy(y_hbm.at[src_v.at[i]], rows.at[b], gsem.at[b])

    def fire_scatter(i, b):
      pltpu.async_copy(rows.at[b], acc.at[dst_v.at[i]], ssem.at[b], add=True)

    def wait_gather(b):
      pltpu.make_async_copy(y_hbm.at[src_v.at[0]], rows.at[b],
                            gsem.at[b]).wait()

    def wait_scatter(b):
      pltpu.make_async_copy(rows.at[b], acc.at[dst_v.at[0]],
                            ssem.at[b]).wait()

    for k in range(PREFETCH):
      fire_gather(k, k)

    plsc.subcore_barrier()

    for i in range(PREFETCH):
      fire_gather(i + PREFETCH, (i + PREFETCH) % NBUF)
      wait_gather(i % NBUF)
      fire_scatter(i, i % NBUF)

    @pl.loop(PREFETCH, CHUNKS_PER_TILE - PREFETCH)
    def _(i):
      b = lax.rem(i, NBUF)
      bn = lax.rem(i + PREFETCH, NBUF)
      wait_scatter(bn)
      fire_gather(i + PREFETCH, bn)
      wait_gather(b)
      fire_scatter(i, b)

    for i in range(CHUNKS_PER_TILE - PREFETCH, CHUNKS_PER_TILE):
      b = i % NBUF
      wait_gather(b)
      fire_scatter(i, b)

    for i in range(CHUNKS_PER_TILE - NBUF, CHUNKS_PER_TILE):
      wait_scatter(i % NBUF)

    plsc.subcore_barrier()
    pltpu.sync_copy(acc.at[pl.ds(row0, ROWS_Z)],
                    out_hbm.at[c].at[pl.ds(row0, ROWS_Z)])

  return seg_kernel(y, ei)


def _tc_project(x, W0, ei):
  def body(x_ref, w_ref, ei_ref, o_ref):
    del ei_ref
    d = jnp.dot(x_ref[...], w_ref[...],
                preferred_element_type=jnp.float32,
                precision=lax.Precision.HIGHEST)
    o_ref[...] = jnp.concatenate(
        [d, jnp.zeros((NP - N, H), jnp.float32)], axis=0)
  return pl.pallas_call(
      body,
      in_specs=[pl.BlockSpec(memory_space=pltpu.MemorySpace.VMEM),
                pl.BlockSpec(memory_space=pltpu.MemorySpace.VMEM),
                pl.BlockSpec(memory_space=pltpu.MemorySpace.HBM)],
      out_shape=jax.ShapeDtypeStruct((NP, H), jnp.float32))(x, W0, ei)


def _row_mask(v):
  rows = lax.broadcasted_iota(jnp.int32, (FLAT, 1), 0)
  return jnp.where(rows < FREAL, v, 0.0)


def _tc_combine_project(acc, y, Wbig, bias128):
  def body(a_ref, y_ref, w_ref, b_ref, o_ref):
    pre = a_ref[0] + a_ref[1] + (1.0 + EPS) * y_ref[...]
    h = _row_mask(jnp.maximum(pre + b_ref[...], 0.0))
    o_ref[...] = jnp.dot(h, w_ref[...],
                         preferred_element_type=jnp.float32,
                         precision=lax.Precision.HIGHEST)
  return pl.pallas_call(
      body, out_shape=jax.ShapeDtypeStruct((FLAT, 128), jnp.float32))(
          acc, y, Wbig, bias128)


def _tc_finalize(acc, y, WoutBig, bias128):
  def body(a_ref, y_ref, w_ref, b_ref, o_ref):
    pre = a_ref[0] + a_ref[1] + (1.0 + EPS) * y_ref[...]
    h = _row_mask(jnp.maximum(pre + b_ref[...], 0.0))
    pooled = jnp.sum(h, axis=0, keepdims=True)
    o_ref[...] = jnp.dot(pooled, w_ref[...],
                         preferred_element_type=jnp.float32,
                         precision=lax.Precision.HIGHEST)
  return pl.pallas_call(
      body, out_shape=jax.ShapeDtypeStruct((1, C), jnp.float32))(
          acc, y, WoutBig, bias128)


def kernel(x, edge_index, W0, W1, W_out, bn_scale, bn_bias, bn_mean, bn_var):
  ei = edge_index.astype(jnp.int32).reshape(
      2, NUM_TILES, CHUNKS_PER_TILE, CHUNK)
  s = (bn_scale * lax.rsqrt(bn_var + BN_EPS)).astype(jnp.float32)
  cvec = bn_bias - bn_mean * s
  bias128 = jnp.tile(cvec.reshape(1, H), (1, 4))
  W0f = W0 * s[None, :]
  W1f = W1 * s[None, :]
  W1big = jnp.kron(jnp.eye(4, dtype=jnp.float32), W1f)
  WoutBig = jnp.tile(W_out, (4, 1))

  y0p = _tc_project(x, W0f, ei)
  acc0 = _sc_segment_sum(y0p, ei)
  y1 = _tc_combine_project(
      acc0.reshape(NUM_CORES, FLAT, 128), y0p.reshape(FLAT, 128),
      W1big, bias128)
  acc1 = _sc_segment_sum(y1.reshape(NP, H), ei)
  out = _tc_finalize(
      acc1.reshape(NUM_CORES, FLAT, 128), y1, WoutBig, bias128)
  return out.reshape(C)


# speedup vs baseline: 1.0460x; 1.0107x over previous
import functools

import jax
import jax.numpy as jnp
from jax import lax
from jax.experimental import pallas as pl
from jax.experimental.pallas import tpu as pltpu
from jax.experimental.pallas import tpu_sc as plsc

N = 10000
E = 320000
D = 128
H = 32
C = 64
EPS = 0.1
BN_EPS = 1e-5

NUM_CORES = 2
NUM_SUBCORES = 16
NUM_TILES = NUM_CORES * NUM_SUBCORES

EDGES_PER_TILE = E // NUM_TILES
CHUNK = 125
CHUNKS_PER_TILE = EDGES_PER_TILE // CHUNK
NBUF = 12
PREFETCH = 6

NP = 10016
FLAT = NP * H // 128
FREAL = N * H // 128
ROWS_Z = NP // NUM_SUBCORES


def _sc_segment_sum(y, ei):
  mesh = plsc.VectorSubcoreMesh(
      core_axis_name="c", subcore_axis_name="s",
      num_cores=NUM_CORES, num_subcores=NUM_SUBCORES)

  @functools.partial(
      pl.kernel,
      out_type=jax.ShapeDtypeStruct((NUM_CORES, NP, H), jnp.float32),
      mesh=mesh,
      scratch_types=[
          pltpu.VMEM((CHUNKS_PER_TILE, CHUNK), jnp.int32),
          pltpu.VMEM((CHUNKS_PER_TILE, CHUNK), jnp.int32),
          pltpu.VMEM((NBUF, CHUNK, H), jnp.float32),
          pltpu.VMEM((ROWS_Z, H), jnp.float32),
          pltpu.VMEM_SHARED((NP, H), jnp.float32),
          pltpu.SemaphoreType.DMA((NBUF,)),
          pltpu.SemaphoreType.DMA((NBUF,)),
          pltpu.SemaphoreType.DMA,
      ],
      compiler_params=pltpu.CompilerParams(use_tc_tiling_on_sc=False),
  )
  def seg_kernel(y_hbm, ei_hbm, out_hbm,
                 src_v, dst_v, rows, zbuf, acc, gsem, ssem, isem):
    c = lax.axis_index("c")
    s = lax.axis_index("s")
    wid = c * NUM_SUBCORES + s

    i_src = pltpu.async_copy(ei_hbm.at[0].at[wid], src_v, isem)
    i_dst = pltpu.async_copy(ei_hbm.at[1].at[wid], dst_v, isem)

    zero16 = jnp.zeros((16,), jnp.float32)

    @pl.loop(0, ROWS_Z)
    def _(r):
      zbuf[r, pl.ds(0, 16)] = zero16
      zbuf[r, pl.ds(16, 16)] = zero16

    row0 = s * ROWS_Z
    pltpu.sync_copy(zbuf, acc.at[pl.ds(row0, ROWS_Z)])

    i_src.wait()
    i_dst.wait()

    def fire_gather(i, b):
      pltpu.async_copy(y_hbm.at[src_v.at[i]], rows.at[b], gsem.at[b])

    def fire_scatter(i, b):
      pltpu.async_copy(rows.at[b], acc.at[dst_v.at[i]], ssem.at[b], add=True)

    def wait_gather(b):
      pltpu.make_async_copy(y_hbm.at[src_v.at[0]], rows.at[b],
                            gsem.at[b]).wait()

    def wait_scatter(b):
      pltpu.make_async_copy(rows.at[b], acc.at[dst_v.at[0]],
                            ssem.at[b]).wait()

    for k in range(PREFETCH):
      fire_gather(k, k)

    plsc.subcore_barrier()

    for i in range(PREFETCH):
      fire_gather(i + PREFETCH, (i + PREFETCH) % NBUF)
      wait_gather(i % NBUF)
      fire_scatter(i, i % NBUF)

    @pl.loop(PREFETCH, CHUNKS_PER_TILE - PREFETCH)
    def _(i):
      b = lax.rem(i, NBUF)
      bn = lax.rem(i + PREFETCH, NBUF)
      wait_scatter(bn)
      fire_gather(i + PREFETCH, bn)
      wait_gather(b)
      fire_scatter(i, b)

    for i in range(CHUNKS_PER_TILE - PREFETCH, CHUNKS_PER_TILE):
      b = i % NBUF
      wait_gather(b)
      fire_scatter(i, b)

    for i in range(CHUNKS_PER_TILE - NBUF, CHUNKS_PER_TILE):
      wait_scatter(i % NBUF)

    plsc.subcore_barrier()
    pltpu.sync_copy(acc.at[pl.ds(row0, ROWS_Z)],
                    out_hbm.at[c].at[pl.ds(row0, ROWS_Z)])

  return seg_kernel(y, ei)


def _tc_project(x, W0, ei):
  def body(x_ref, w_ref, ei_ref, o_ref):
    del ei_ref
    d = jnp.dot(x_ref[...], w_ref[...],
                preferred_element_type=jnp.float32,
                precision=lax.Precision.HIGHEST)
    o_ref[...] = jnp.concatenate(
        [d, jnp.zeros((NP - N, H), jnp.float32)], axis=0)
  return pl.pallas_call(
      body,
      in_specs=[pl.BlockSpec(memory_space=pltpu.MemorySpace.VMEM),
                pl.BlockSpec(memory_space=pltpu.MemorySpace.VMEM),
                pl.BlockSpec(memory_space=pltpu.MemorySpace.HBM)],
      out_shape=jax.ShapeDtypeStruct((NP, H), jnp.float32))(x, W0, ei)


def _row_mask(v):
  rows = lax.broadcasted_iota(jnp.int32, (FLAT, 1), 0)
  return jnp.where(rows < FREAL, v, 0.0)


def _tc_combine_project(acc, y, Wbig, bias128):
  def body(a_ref, y_ref, w_ref, b_ref, o_ref):
    pre = a_ref[0] + a_ref[1] + (1.0 + EPS) * y_ref[...]
    h = _row_mask(jnp.maximum(pre + b_ref[...], 0.0))
    o_ref[...] = jnp.dot(h, w_ref[...],
                         preferred_element_type=jnp.float32,
                         precision=lax.Precision.HIGHEST)
  return pl.pallas_call(
      body, out_shape=jax.ShapeDtypeStruct((FLAT, 128), jnp.float32))(
          acc, y, Wbig, bias128)


def _tc_finalize(acc, y, WoutBig, bias128):
  def body(a_ref, y_ref, w_ref, b_ref, o_ref):
    pre = a_ref[0] + a_ref[1] + (1.0 + EPS) * y_ref[...]
    h = _row_mask(jnp.maximum(pre + b_ref[...], 0.0))
    pooled = jnp.sum(h, axis=0, keepdims=True)
    o_ref[...] = jnp.dot(pooled, w_ref[...],
                         preferred_element_type=jnp.float32,
                         precision=lax.Precision.HIGHEST)
  return pl.pallas_call(
      body, out_shape=jax.ShapeDtypeStruct((1, C), jnp.float32))(
          acc, y, WoutBig, bias128)


def kernel(x, edge_index, W0, W1, W_out, bn_scale, bn_bias, bn_mean, bn_var):
  ei = edge_index.astype(jnp.int32).reshape(
      2, NUM_TILES, CHUNKS_PER_TILE, CHUNK)
  s = (bn_scale * lax.rsqrt(bn_var + BN_EPS)).astype(jnp.float32)
  cvec = bn_bias - bn_mean * s
  bias128 = jnp.tile(cvec.reshape(1, H), (1, 4))
  W0f = W0 * s[None, :]
  W1f = W1 * s[None, :]
  W1big = jnp.kron(jnp.eye(4, dtype=jnp.float32), W1f)
  WoutBig = jnp.tile(W_out, (4, 1))

  y0p = _tc_project(x, W0f, ei)
  acc0 = _sc_segment_sum(y0p, ei)
  y1 = _tc_combine_project(
      acc0.reshape(NUM_CORES, FLAT, 128), y0p.reshape(FLAT, 128),
      W1big, bias128)
  acc1 = _sc_segment_sum(y1.reshape(NP, H), ei)
  out = _tc_finalize(
      acc1.reshape(NUM_CORES, FLAT, 128), y1, WoutBig, bias128)
  return out.reshape(C)
